# baseline (speedup 1.0000x reference)
import functools

import jax
import jax.numpy as jnp
from jax import lax
from jax.experimental import pallas as pl
from jax.experimental.pallas import tpu as pltpu
from jax.experimental.pallas import tpu_sc as plsc

N = 10000
E = 320000
D = 128

NC = 2
NS = 16
LANES = 16
NP = 10240
C = 128
EP = 327680
ET = EP // (NC * NS)
NCH = ET // C
RPS = NP // NS
NBLK = 5
BCH = NCH // NBLK
NB = 2
NR = 2

_MESH = dict(core_axis_name="c", subcore_axis_name="s",
             num_cores=NC, num_subcores=NS)


def _sc_degrees(e6, zeros):
    mesh = plsc.VectorSubcoreMesh(**_MESH)

    @functools.partial(
        pl.kernel,
        out_type=jax.ShapeDtypeStruct((NC, 2, NP), jnp.float32),
        mesh=mesh,
        scratch_types=[
            pltpu.VMEM((2, NB, BCH, C), jnp.int32),
            pltpu.VMEM((C,), jnp.float32),
            pltpu.VMEM_SHARED((NP,), jnp.float32),
            pltpu.VMEM_SHARED((NP,), jnp.float32),
            pltpu.SemaphoreType.DMA,
        ],
    )
    def k(e_hbm, z_hbm, out_hbm, idx, ones, acc_o, acc_i, semi):
        core = lax.axis_index("c")
        sub = lax.axis_index("s")

        @pl.loop(0, C // LANES)
        def _(r):
            ones[pl.ds(r * LANES, LANES)] = jnp.ones((LANES,), jnp.float32)

        pltpu.sync_copy(e_hbm.at[0, core, sub, 0], idx.at[0, 0])
        pltpu.sync_copy(e_hbm.at[1, core, sub, 0], idx.at[1, 0])
        pltpu.sync_copy(z_hbm, acc_o.at[pl.ds(sub * RPS, RPS)])
        pltpu.sync_copy(z_hbm, acc_i.at[pl.ds(sub * RPS, RPS)])
        plsc.subcore_barrier()

        for b in range(NBLK):
            sidx = idx.at[0, b % NB]
            didx = idx.at[1, b % NB]
            if b + 1 < NBLK:
                pltpu.async_copy(e_hbm.at[0, core, sub, b + 1],
                                 idx.at[0, (b + 1) % NB], semi)
                pltpu.async_copy(e_hbm.at[1, core, sub, b + 1],
                                 idx.at[1, (b + 1) % NB], semi)

            @pl.loop(0, BCH)
            def _(j):
                pltpu.sync_copy(ones, acc_o.at[sidx.at[j]], add=True)
                pltpu.sync_copy(ones, acc_i.at[didx.at[j]], add=True)

            if b + 1 < NBLK:
                pltpu.make_async_copy(e_hbm.at[0, core, sub, b + 1],
                                      idx.at[0, (b + 1) % NB], semi).wait()
                pltpu.make_async_copy(e_hbm.at[1, core, sub, b + 1],
                                      idx.at[1, (b + 1) % NB], semi).wait()

        plsc.subcore_barrier()
        pltpu.sync_copy(acc_o.at[pl.ds(sub * RPS, RPS)],
                        out_hbm.at[core, 0, pl.ds(sub * RPS, RPS)])
        pltpu.sync_copy(acc_i.at[pl.ds(sub * RPS, RPS)],
                        out_hbm.at[core, 1, pl.ds(sub * RPS, RPS)])

    return k(e6, zeros)


def _sc_edge_pass(h, e6, zeros):
    mesh = plsc.VectorSubcoreMesh(**_MESH)

    @functools.partial(
        pl.kernel,
        out_type=jax.ShapeDtypeStruct((NC, NP, D), jnp.float32),
        mesh=mesh,
        scratch_types=[
            pltpu.VMEM((2, NB, BCH, C), jnp.int32),
            pltpu.VMEM((NR, C, D), jnp.float32),
            pltpu.VMEM_SHARED((NP, D), jnp.float32),
            [pltpu.SemaphoreType.DMA] * NR,
            pltpu.SemaphoreType.DMA,
        ],
    )
    def k(h_hbm, e_hbm, z_hbm, out_hbm,
          idx, rows, acc, sems, semi):
        core = lax.axis_index("c")
        sub = lax.axis_index("s")

        pltpu.sync_copy(e_hbm.at[0, core, sub, 0], idx.at[0, 0])
        pltpu.sync_copy(e_hbm.at[1, core, sub, 0], idx.at[1, 0])
        pltpu.sync_copy(z_hbm, acc.at[pl.ds(sub * RPS, RPS)])
        plsc.subcore_barrier()

        @pl.loop(0, NBLK)
        def _(b):
            sidx = idx.at[0, b % NB]
            didx = idx.at[1, b % NB]

            @pl.when(b + 1 < NBLK)
            def _():
                pltpu.async_copy(e_hbm.at[0, core, sub, b + 1],
                                 idx.at[0, (b + 1) % NB], semi)
                pltpu.async_copy(e_hbm.at[1, core, sub, b + 1],
                                 idx.at[1, (b + 1) % NB], semi)

            for k0 in range(NR - 1):
                pltpu.async_copy(h_hbm.at[sidx.at[k0]], rows.at[k0],
                                 sems[k0])

            @pl.loop(0, BCH // NR)
            def _(t):
                c0 = t * NR
                for k in range(NR):
                    c = c0 + k
                    pltpu.make_async_copy(h_hbm.at[sidx.at[c]], rows.at[k],
                                          sems[k]).wait()
                    pltpu.sync_copy(rows.at[k], acc.at[didx.at[c]],
                                    add=True)
                    nk = (k + NR - 1) % NR
                    nc = c + NR - 1

                    @pl.when(nc < BCH)
                    def _():
                        pltpu.async_copy(h_hbm.at[sidx.at[nc]], rows.at[nk],
                                         sems[nk])

            for k in range(BCH - BCH // NR * NR):
                c = BCH // NR * NR + k
                pltpu.make_async_copy(h_hbm.at[sidx.at[c]], rows.at[k],
                                      sems[k]).wait()
                pltpu.sync_copy(rows.at[k], acc.at[didx.at[c]], add=True)

            @pl.when(b + 1 < NBLK)
            def _():
                pltpu.make_async_copy(e_hbm.at[0, core, sub, b + 1],
                                      idx.at[0, (b + 1) % NB], semi).wait()
                pltpu.make_async_copy(e_hbm.at[1, core, sub, b + 1],
                                      idx.at[1, (b + 1) % NB], semi).wait()

        plsc.subcore_barrier()
        pltpu.sync_copy(acc.at[pl.ds(sub * RPS, RPS)],
                        out_hbm.at[core, pl.ds(sub * RPS, RPS)])

    return k(h, e6, zeros)


_R = 2048
_G = NP // _R


def _nrm(dg):
    return jnp.where(dg > 0, lax.rsqrt(jnp.maximum(dg, 1.0)), 0.0)


_DEG_SPEC = pl.BlockSpec((NC, 2, _R), lambda i: (0, 0, i))


def _tc_mm(x, w):
    def body(x_ref, w_ref, o_ref):
        o_ref[...] = jnp.dot(x_ref[...], w_ref[...],
                             preferred_element_type=jnp.float32)

    return pl.pallas_call(
        body,
        grid=(_G,),
        in_specs=[pl.BlockSpec((_R, D), lambda i: (i, 0)),
                  pl.BlockSpec((D, D), lambda i: (0, 0))],
        out_specs=pl.BlockSpec((_R, D), lambda i: (i, 0)),
        out_shape=jax.ShapeDtypeStruct((NP, D), jnp.float32),
    )(x, w)


def _tc_norm_scale(degp, u):
    def body(dp_ref, u_ref, h_ref):
        ns = _nrm(dp_ref[0, 0] + dp_ref[1, 0]).reshape(_R, 1)
        h_ref[...] = u_ref[...] * ns

    return pl.pallas_call(
        body,
        grid=(_G,),
        in_specs=[_DEG_SPEC,
                  pl.BlockSpec((_R, D), lambda i: (i, 0))],
        out_specs=pl.BlockSpec((_R, D), lambda i: (i, 0)),
        out_shape=jax.ShapeDtypeStruct((NP, D), jnp.float32),
    )(degp, u)


def _tc_layer2(p, degp, b1, gamma, beta, w):
    def body(p_ref, dp_ref, b_ref, g_ref, bt_ref, w_ref, o_ref,
             h_scr, st_scr):
        i = pl.program_id(0)

        @pl.when(i < _G)
        def _():
            nd = _nrm(dp_ref[0, 1] + dp_ref[1, 1]).reshape(_R, 1)
            h = (p_ref[0] + p_ref[1]) * nd + b_ref[...]
            h_scr[pl.ds(i * _R, _R), :] = h
            row = i * _R + lax.broadcasted_iota(jnp.int32, (_R, 1), 0)
            hm = jnp.where(row < N, h, 0.0)
            st = jnp.concatenate(
                [jnp.sum(hm, axis=0, keepdims=True),
                 jnp.sum(hm * hm, axis=0, keepdims=True)], axis=0)

            @pl.when(i == 0)
            def _():
                st_scr[...] = st

            @pl.when(i != 0)
            def _():
                st_scr[...] += st

        @pl.when(i >= _G)
        def _():
            mean = st_scr[0:1] / N
            var = st_scr[1:2] / N - mean * mean
            a = g_ref[...] * lax.rsqrt(var + 1e-5)
            c = bt_ref[...] - mean * a
            ns = _nrm(dp_ref[0, 0] + dp_ref[1, 0]).reshape(_R, 1)
            h = h_scr[pl.ds((i - _G) * _R, _R), :]
            hh = jnp.maximum(h * a + c, 0.0) * ns
            o_ref[...] = jnp.dot(hh, w_ref[...],
                                 preferred_element_type=jnp.float32)

    return pl.pallas_call(
        body,
        grid=(2 * _G,),
        in_specs=[
            pl.BlockSpec((NC, _R, D),
                         lambda i: (0, jnp.where(i < _G, i, _G - 1), 0)),
            pl.BlockSpec((NC, 2, _R), lambda i: (0, 0, i % _G)),
            pl.BlockSpec((1, D), lambda i: (0, 0)),
            pl.BlockSpec((1, D), lambda i: (0, 0)),
            pl.BlockSpec((1, D), lambda i: (0, 0)),
            pl.BlockSpec((D, D), lambda i: (0, 0)),
        ],
        out_specs=pl.BlockSpec((_R, D), lambda i: (i % _G, 0)),
        out_shape=jax.ShapeDtypeStruct((NP, D), jnp.float32),
        scratch_shapes=[pltpu.VMEM((NP, D), jnp.float32),
                        pltpu.VMEM((2, D), jnp.float32)],
    )(p, degp, b1, gamma, beta, w)


def _tc_final(q, degp, b2):
    def body(q_ref, dp_ref, b_ref, o_ref):
        nd = _nrm(dp_ref[0, 1] + dp_ref[1, 1]).reshape(_R, 1)
        o_ref[...] = (q_ref[0] + q_ref[1]) * nd + b_ref[...]

    return pl.pallas_call(
        body,
        grid=(_G,),
        in_specs=[pl.BlockSpec((NC, _R, D), lambda i: (0, i, 0)),
                  _DEG_SPEC,
                  pl.BlockSpec((1, D), lambda i: (0, 0))],
        out_specs=pl.BlockSpec((_R, D), lambda i: (i, 0)),
        out_shape=jax.ShapeDtypeStruct((N, D), jnp.float32),
    )(q, degp, b2)


def kernel(x, edge_index, W1, b1, gamma, beta, W2, b2):
    pad_e = jnp.broadcast_to(jnp.array([[N], [NP - 1]], jnp.int32),
                             (2, EP - E))
    e6 = jnp.concatenate([edge_index, pad_e],
                         axis=1).reshape(2, NC, NS, NBLK, BCH, C)
    zeros = jnp.zeros((RPS, D), jnp.float32)
    zeros1 = jnp.zeros((RPS,), jnp.float32)
    x_p = jnp.concatenate(
        [x, jnp.zeros((NP - N, D), jnp.float32)], axis=0)

    u = _tc_mm(x_p, W1)
    degp = _sc_degrees(e6, zeros1)
    h1 = _tc_norm_scale(degp, u)

    p = _sc_edge_pass(h1, e6, zeros)
    h2 = _tc_layer2(p, degp, b1.reshape(1, D), gamma.reshape(1, D),
                    beta.reshape(1, D), W2)
    q = _sc_edge_pass(h2, e6, zeros)
    return _tc_final(q, degp, b2.reshape(1, D))


# speedup vs baseline: 2.6334x; 2.6334x over previous
import functools

import jax
import jax.numpy as jnp
from jax import lax
from jax.experimental import pallas as pl
from jax.experimental.pallas import tpu as pltpu
from jax.experimental.pallas import tpu_sc as plsc

N = 10000
E = 320000
D = 128

NC = 2
NS = 16
LANES = 16
NP = 10240
C = 128
EP = 327680
ET = EP // (NC * NS)
NCH = ET // C
RPS = NP // NS
NBLK = 5
BCH = NCH // NBLK
NB = 2
NR = 2

_MESH = dict(core_axis_name="c", subcore_axis_name="s",
             num_cores=NC, num_subcores=NS)


def _sc_degrees(e6, zeros):
    mesh = plsc.VectorSubcoreMesh(**_MESH)

    @functools.partial(
        pl.kernel,
        out_type=jax.ShapeDtypeStruct((NC, 2, NP), jnp.float32),
        mesh=mesh,
        scratch_types=[
            pltpu.VMEM((2, NB, BCH, C), jnp.int32),
            pltpu.VMEM((C,), jnp.float32),
            pltpu.VMEM_SHARED((NP,), jnp.float32),
            pltpu.VMEM_SHARED((NP,), jnp.float32),
            pltpu.SemaphoreType.DMA,
        ],
    )
    def k(e_hbm, z_hbm, out_hbm, idx, ones, acc_o, acc_i, semi):
        core = lax.axis_index("c")
        sub = lax.axis_index("s")

        @pl.loop(0, C // LANES)
        def _(r):
            ones[pl.ds(r * LANES, LANES)] = jnp.ones((LANES,), jnp.float32)

        pltpu.sync_copy(e_hbm.at[0, core, sub, 0], idx.at[0, 0])
        pltpu.sync_copy(e_hbm.at[1, core, sub, 0], idx.at[1, 0])
        pltpu.sync_copy(z_hbm, acc_o.at[pl.ds(sub * RPS, RPS)])
        pltpu.sync_copy(z_hbm, acc_i.at[pl.ds(sub * RPS, RPS)])
        plsc.subcore_barrier()

        for b in range(NBLK):
            sidx = idx.at[0, b % NB]
            didx = idx.at[1, b % NB]
            if b + 1 < NBLK:
                pltpu.async_copy(e_hbm.at[0, core, sub, b + 1],
                                 idx.at[0, (b + 1) % NB], semi)
                pltpu.async_copy(e_hbm.at[1, core, sub, b + 1],
                                 idx.at[1, (b + 1) % NB], semi)

            @pl.loop(0, BCH)
            def _(j):
                pltpu.sync_copy(ones, acc_o.at[sidx.at[j]], add=True)
                pltpu.sync_copy(ones, acc_i.at[didx.at[j]], add=True)

            if b + 1 < NBLK:
                pltpu.make_async_copy(e_hbm.at[0, core, sub, b + 1],
                                      idx.at[0, (b + 1) % NB], semi).wait()
                pltpu.make_async_copy(e_hbm.at[1, core, sub, b + 1],
                                      idx.at[1, (b + 1) % NB], semi).wait()

        plsc.subcore_barrier()
        pltpu.sync_copy(acc_o.at[pl.ds(sub * RPS, RPS)],
                        out_hbm.at[core, 0, pl.ds(sub * RPS, RPS)])
        pltpu.sync_copy(acc_i.at[pl.ds(sub * RPS, RPS)],
                        out_hbm.at[core, 1, pl.ds(sub * RPS, RPS)])

    return k(e6, zeros)


def _sc_edge_pass(h, e6, zeros):
    mesh = plsc.VectorSubcoreMesh(**_MESH)

    @functools.partial(
        pl.kernel,
        out_type=jax.ShapeDtypeStruct((NC, NP, D), jnp.float32),
        mesh=mesh,
        scratch_types=[
            pltpu.VMEM((2, NB, BCH, C), jnp.int32),
            pltpu.VMEM((NR, C, D), jnp.float32),
            pltpu.VMEM_SHARED((NP, D), jnp.float32),
            [pltpu.SemaphoreType.DMA] * NR,
            pltpu.SemaphoreType.DMA,
        ],
    )
    def k(h_hbm, e_hbm, z_hbm, out_hbm,
          idx, rows, acc, sems, semi):
        core = lax.axis_index("c")
        sub = lax.axis_index("s")

        pltpu.sync_copy(e_hbm.at[0, core, sub, 0], idx.at[0, 0])
        pltpu.sync_copy(e_hbm.at[1, core, sub, 0], idx.at[1, 0])
        pltpu.sync_copy(z_hbm, acc.at[pl.ds(sub * RPS, RPS)])
        plsc.subcore_barrier()

        @pl.loop(0, NBLK)
        def _(b):
            sidx = idx.at[0, b % NB]
            didx = idx.at[1, b % NB]

            @pl.when(b + 1 < NBLK)
            def _():
                pltpu.async_copy(e_hbm.at[0, core, sub, b + 1],
                                 idx.at[0, (b + 1) % NB], semi)
                pltpu.async_copy(e_hbm.at[1, core, sub, b + 1],
                                 idx.at[1, (b + 1) % NB], semi)

            for k0 in range(NR - 1):
                pltpu.async_copy(h_hbm.at[sidx.at[k0]], rows.at[k0],
                                 sems[k0])

            @pl.loop(0, BCH // NR)
            def _(t):
                c0 = t * NR
                for k in range(NR):
                    c = c0 + k
                    pltpu.make_async_copy(h_hbm.at[sidx.at[c]], rows.at[k],
                                          sems[k]).wait()
                    pltpu.sync_copy(rows.at[k], acc.at[didx.at[c]],
                                    add=True)
                    nk = (k + NR - 1) % NR
                    nc = c + NR - 1

                    @pl.when(nc < BCH)
                    def _():
                        pltpu.async_copy(h_hbm.at[sidx.at[nc]], rows.at[nk],
                                         sems[nk])

            for k in range(BCH - BCH // NR * NR):
                c = BCH // NR * NR + k
                pltpu.make_async_copy(h_hbm.at[sidx.at[c]], rows.at[k],
                                      sems[k]).wait()
                pltpu.sync_copy(rows.at[k], acc.at[didx.at[c]], add=True)

            @pl.when(b + 1 < NBLK)
            def _():
                pltpu.make_async_copy(e_hbm.at[0, core, sub, b + 1],
                                      idx.at[0, (b + 1) % NB], semi).wait()
                pltpu.make_async_copy(e_hbm.at[1, core, sub, b + 1],
                                      idx.at[1, (b + 1) % NB], semi).wait()

        plsc.subcore_barrier()
        pltpu.sync_copy(acc.at[pl.ds(sub * RPS, RPS)],
                        out_hbm.at[core, pl.ds(sub * RPS, RPS)])

    return k(h, e6, zeros)


_R = 2048
_G = NP // _R


def _nrm(dg):
    return jnp.where(dg > 0, lax.rsqrt(jnp.maximum(dg, 1.0)), 0.0)


_DEG_SPEC = pl.BlockSpec((NC, 2, _R), lambda i: (0, 0, i))


def _tc_mm(x, w):
    def body(x_ref, w_ref, o_ref):
        o_ref[...] = jnp.dot(x_ref[...], w_ref[...],
                             preferred_element_type=jnp.float32)

    return pl.pallas_call(
        body,
        grid=(_G,),
        in_specs=[pl.BlockSpec((_R, D), lambda i: (i, 0)),
                  pl.BlockSpec((D, D), lambda i: (0, 0))],
        out_specs=pl.BlockSpec((_R, D), lambda i: (i, 0)),
        out_shape=jax.ShapeDtypeStruct((NP, D), jnp.float32),
    )(x, w)


def _tc_norm_scale(degp, u):
    def body(dp_ref, u_ref, h_ref):
        ns = _nrm(dp_ref[0, 0] + dp_ref[1, 0]).reshape(_R, 1)
        h_ref[...] = u_ref[...] * ns

    return pl.pallas_call(
        body,
        grid=(_G,),
        in_specs=[_DEG_SPEC,
                  pl.BlockSpec((_R, D), lambda i: (i, 0))],
        out_specs=pl.BlockSpec((_R, D), lambda i: (i, 0)),
        out_shape=jax.ShapeDtypeStruct((NP, D), jnp.float32),
    )(degp, u)


def _tc_layer2(p, degp, b1, gamma, beta, w):
    def body(p_ref, dp_ref, b_ref, g_ref, bt_ref, w_ref, o_ref,
             h_scr, st_scr):
        i = pl.program_id(0)

        @pl.when(i < _G)
        def _():
            nd = _nrm(dp_ref[0, 1] + dp_ref[1, 1]).reshape(_R, 1)
            h = (p_ref[0] + p_ref[1]) * nd + b_ref[...]
            h_scr[pl.ds(i * _R, _R), :] = h
            row = i * _R + lax.broadcasted_iota(jnp.int32, (_R, 1), 0)
            hm = jnp.where(row < N, h, 0.0)
            st = jnp.concatenate(
                [jnp.sum(hm, axis=0, keepdims=True),
                 jnp.sum(hm * hm, axis=0, keepdims=True)], axis=0)

            @pl.when(i == 0)
            def _():
                st_scr[...] = st

            @pl.when(i != 0)
            def _():
                st_scr[...] += st

        @pl.when(i >= _G)
        def _():
            mean = st_scr[0:1] / N
            var = st_scr[1:2] / N - mean * mean
            a = g_ref[...] * lax.rsqrt(var + 1e-5)
            c = bt_ref[...] - mean * a
            ns = _nrm(dp_ref[0, 0] + dp_ref[1, 0]).reshape(_R, 1)
            h = h_scr[pl.ds((i - _G) * _R, _R), :]
            hh = jnp.maximum(h * a + c, 0.0) * ns
            o_ref[...] = jnp.dot(hh, w_ref[...],
                                 preferred_element_type=jnp.float32)

    return pl.pallas_call(
        body,
        grid=(2 * _G,),
        in_specs=[
            pl.BlockSpec((NC, _R, D),
                         lambda i: (0, jnp.where(i < _G, i, _G - 1), 0)),
            pl.BlockSpec((NC, 2, _R), lambda i: (0, 0, i % _G)),
            pl.BlockSpec((1, D), lambda i: (0, 0)),
            pl.BlockSpec((1, D), lambda i: (0, 0)),
            pl.BlockSpec((1, D), lambda i: (0, 0)),
            pl.BlockSpec((D, D), lambda i: (0, 0)),
        ],
        out_specs=pl.BlockSpec((_R, D), lambda i: (i % _G, 0)),
        out_shape=jax.ShapeDtypeStruct((NP, D), jnp.float32),
        scratch_shapes=[pltpu.VMEM((NP, D), jnp.float32),
                        pltpu.VMEM((2, D), jnp.float32)],
    )(p, degp, b1, gamma, beta, w)


def _tc_final(q, degp, b2):
    def body(q_ref, dp_ref, b_ref, o_ref):
        nd = _nrm(dp_ref[0, 1] + dp_ref[1, 1]).reshape(_R, 1)
        o_ref[...] = (q_ref[0] + q_ref[1]) * nd + b_ref[...]

    return pl.pallas_call(
        body,
        grid=(_G,),
        in_specs=[pl.BlockSpec((NC, _R, D), lambda i: (0, i, 0)),
                  _DEG_SPEC,
                  pl.BlockSpec((1, D), lambda i: (0, 0))],
        out_specs=pl.BlockSpec((_R, D), lambda i: (i, 0)),
        out_shape=jax.ShapeDtypeStruct((N, D), jnp.float32),
    )(q, degp, b2)


def kernel(x, edge_index, W1, b1, gamma, beta, W2, b2):
    trash = N + jnp.arange(EP - E, dtype=jnp.int32) % (NP - N)
    e6 = jnp.concatenate([edge_index, jnp.stack([trash, trash])],
                         axis=1).reshape(2, NC, NS, NBLK, BCH, C)
    zeros = jnp.zeros((RPS, D), jnp.float32)
    zeros1 = jnp.zeros((RPS,), jnp.float32)
    x_p = jnp.concatenate(
        [x, jnp.zeros((NP - N, D), jnp.float32)], axis=0)

    u = _tc_mm(x_p, W1)
    degp = _sc_degrees(e6, zeros1)
    h1 = _tc_norm_scale(degp, u)

    p = _sc_edge_pass(h1, e6, zeros)
    h2 = _tc_layer2(p, degp, b1.reshape(1, D), gamma.reshape(1, D),
                    beta.reshape(1, D), W2)
    q = _sc_edge_pass(h2, e6, zeros)
    return _tc_final(q, degp, b2.reshape(1, D))


# speedup vs baseline: 3.4749x; 1.3195x over previous
import functools

import jax
import jax.numpy as jnp
from jax import lax
from jax.experimental import pallas as pl
from jax.experimental.pallas import tpu as pltpu
from jax.experimental.pallas import tpu_sc as plsc

N = 10000
E = 320000
D = 128

NC = 2
NS = 16
LANES = 16
NP = 10240
C = 80
ET = E // (NC * NS)
NCH = ET // C
RPS = NP // NS
NBLK = 5
BCH = NCH // NBLK
NB = 2
NR = 3

_MESH = dict(core_axis_name="c", subcore_axis_name="s",
             num_cores=NC, num_subcores=NS)


def _sc_degrees(e6, zeros):
    mesh = plsc.VectorSubcoreMesh(**_MESH)

    @functools.partial(
        pl.kernel,
        out_type=jax.ShapeDtypeStruct((NC, 2, NP), jnp.float32),
        mesh=mesh,
        scratch_types=[
            pltpu.VMEM((2, NB, BCH, C), jnp.int32),
            pltpu.VMEM((C,), jnp.float32),
            pltpu.VMEM_SHARED((NP,), jnp.float32),
            pltpu.VMEM_SHARED((NP,), jnp.float32),
            pltpu.SemaphoreType.DMA,
        ],
    )
    def k(e_hbm, z_hbm, out_hbm, idx, ones, acc_o, acc_i, semi):
        core = lax.axis_index("c")
        sub = lax.axis_index("s")

        @pl.loop(0, C // LANES)
        def _(r):
            ones[pl.ds(r * LANES, LANES)] = jnp.ones((LANES,), jnp.float32)

        pltpu.sync_copy(e_hbm.at[0, core, sub, 0], idx.at[0, 0])
        pltpu.sync_copy(e_hbm.at[1, core, sub, 0], idx.at[1, 0])
        pltpu.sync_copy(z_hbm, acc_o.at[pl.ds(sub * RPS, RPS)])
        pltpu.sync_copy(z_hbm, acc_i.at[pl.ds(sub * RPS, RPS)])
        plsc.subcore_barrier()

        for b in range(NBLK):
            sidx = idx.at[0, b % NB]
            didx = idx.at[1, b % NB]
            if b + 1 < NBLK:
                pltpu.async_copy(e_hbm.at[0, core, sub, b + 1],
                                 idx.at[0, (b + 1) % NB], semi)
                pltpu.async_copy(e_hbm.at[1, core, sub, b + 1],
                                 idx.at[1, (b + 1) % NB], semi)

            @pl.loop(0, BCH)
            def _(j):
                pltpu.sync_copy(ones, acc_o.at[sidx.at[j]], add=True)
                pltpu.sync_copy(ones, acc_i.at[didx.at[j]], add=True)

            if b + 1 < NBLK:
                pltpu.make_async_copy(e_hbm.at[0, core, sub, b + 1],
                                      idx.at[0, (b + 1) % NB], semi).wait()
                pltpu.make_async_copy(e_hbm.at[1, core, sub, b + 1],
                                      idx.at[1, (b + 1) % NB], semi).wait()

        plsc.subcore_barrier()
        pltpu.sync_copy(acc_o.at[pl.ds(sub * RPS, RPS)],
                        out_hbm.at[core, 0, pl.ds(sub * RPS, RPS)])
        pltpu.sync_copy(acc_i.at[pl.ds(sub * RPS, RPS)],
                        out_hbm.at[core, 1, pl.ds(sub * RPS, RPS)])

    return k(e6, zeros)


def _sc_edge_pass(h, e6, zeros):
    mesh = plsc.VectorSubcoreMesh(**_MESH)

    @functools.partial(
        pl.kernel,
        out_type=jax.ShapeDtypeStruct((NC, NP, D), jnp.float32),
        mesh=mesh,
        scratch_types=[
            pltpu.VMEM((2, NB, BCH, C), jnp.int32),
            pltpu.VMEM((NR, C, D), jnp.float32),
            pltpu.VMEM_SHARED((NP, D), jnp.float32),
            [pltpu.SemaphoreType.DMA] * NR,
            pltpu.SemaphoreType.DMA,
        ],
    )
    def k(h_hbm, e_hbm, z_hbm, out_hbm,
          idx, rows, acc, sems, semi):
        core = lax.axis_index("c")
        sub = lax.axis_index("s")

        pltpu.sync_copy(e_hbm.at[0, core, sub, 0], idx.at[0, 0])
        pltpu.sync_copy(e_hbm.at[1, core, sub, 0], idx.at[1, 0])
        pltpu.sync_copy(z_hbm, acc.at[pl.ds(sub * RPS, RPS)])
        plsc.subcore_barrier()

        @pl.loop(0, NBLK)
        def _(b):
            sidx = idx.at[0, b % NB]
            didx = idx.at[1, b % NB]

            @pl.when(b + 1 < NBLK)
            def _():
                pltpu.async_copy(e_hbm.at[0, core, sub, b + 1],
                                 idx.at[0, (b + 1) % NB], semi)
                pltpu.async_copy(e_hbm.at[1, core, sub, b + 1],
                                 idx.at[1, (b + 1) % NB], semi)

            for k0 in range(NR - 1):
                pltpu.async_copy(h_hbm.at[sidx.at[k0]], rows.at[k0],
                                 sems[k0])

            @pl.loop(0, BCH // NR)
            def _(t):
                c0 = t * NR
                for k in range(NR):
                    c = c0 + k
                    pltpu.make_async_copy(h_hbm.at[sidx.at[c]], rows.at[k],
                                          sems[k]).wait()
                    pltpu.sync_copy(rows.at[k], acc.at[didx.at[c]],
                                    add=True)
                    nk = (k + NR - 1) % NR
                    nc = c + NR - 1

                    @pl.when(nc < BCH)
                    def _():
                        pltpu.async_copy(h_hbm.at[sidx.at[nc]], rows.at[nk],
                                         sems[nk])

            for k in range(BCH - BCH // NR * NR):
                c = BCH // NR * NR + k
                pltpu.make_async_copy(h_hbm.at[sidx.at[c]], rows.at[k],
                                      sems[k]).wait()
                pltpu.sync_copy(rows.at[k], acc.at[didx.at[c]], add=True)

            @pl.when(b + 1 < NBLK)
            def _():
                pltpu.make_async_copy(e_hbm.at[0, core, sub, b + 1],
                                      idx.at[0, (b + 1) % NB], semi).wait()
                pltpu.make_async_copy(e_hbm.at[1, core, sub, b + 1],
                                      idx.at[1, (b + 1) % NB], semi).wait()

        plsc.subcore_barrier()
        pltpu.sync_copy(acc.at[pl.ds(sub * RPS, RPS)],
                        out_hbm.at[core, pl.ds(sub * RPS, RPS)])

    return k(h, e6, zeros)


_R = 2048
_G = NP // _R


def _nrm(dg):
    return jnp.where(dg > 0, lax.rsqrt(jnp.maximum(dg, 1.0)), 0.0)


_DEG_SPEC = pl.BlockSpec((NC, 2, _R), lambda i: (0, 0, i))


def _tc_mm(x, w):
    def body(x_ref, w_ref, o_ref):
        o_ref[...] = jnp.dot(x_ref[...], w_ref[...],
                             preferred_element_type=jnp.float32)

    return pl.pallas_call(
        body,
        grid=(_G,),
        in_specs=[pl.BlockSpec((_R, D), lambda i: (i, 0)),
                  pl.BlockSpec((D, D), lambda i: (0, 0))],
        out_specs=pl.BlockSpec((_R, D), lambda i: (i, 0)),
        out_shape=jax.ShapeDtypeStruct((NP, D), jnp.float32),
    )(x, w)


def _tc_norm_scale(degp, u):
    def body(dp_ref, u_ref, h_ref):
        ns = _nrm(dp_ref[0, 0] + dp_ref[1, 0]).reshape(_R, 1)
        h_ref[...] = u_ref[...] * ns

    return pl.pallas_call(
        body,
        grid=(_G,),
        in_specs=[_DEG_SPEC,
                  pl.BlockSpec((_R, D), lambda i: (i, 0))],
        out_specs=pl.BlockSpec((_R, D), lambda i: (i, 0)),
        out_shape=jax.ShapeDtypeStruct((NP, D), jnp.float32),
    )(degp, u)


def _tc_layer2(p, degp, b1, gamma, beta, w):
    def body(p_ref, dp_ref, b_ref, g_ref, bt_ref, w_ref, o_ref,
             h_scr, st_scr):
        i = pl.program_id(0)

        @pl.when(i < _G)
        def _():
            nd = _nrm(dp_ref[0, 1] + dp_ref[1, 1]).reshape(_R, 1)
            h = (p_ref[0] + p_ref[1]) * nd + b_ref[...]
            h_scr[pl.ds(i * _R, _R), :] = h
            row = i * _R + lax.broadcasted_iota(jnp.int32, (_R, 1), 0)
            hm = jnp.where(row < N, h, 0.0)
            st = jnp.concatenate(
                [jnp.sum(hm, axis=0, keepdims=True),
                 jnp.sum(hm * hm, axis=0, keepdims=True)], axis=0)

            @pl.when(i == 0)
            def _():
                st_scr[...] = st

            @pl.when(i != 0)
            def _():
                st_scr[...] += st

        @pl.when(i >= _G)
        def _():
            mean = st_scr[0:1] / N
            var = st_scr[1:2] / N - mean * mean
            a = g_ref[...] * lax.rsqrt(var + 1e-5)
            c = bt_ref[...] - mean * a
            ns = _nrm(dp_ref[0, 0] + dp_ref[1, 0]).reshape(_R, 1)
            h = h_scr[pl.ds((i - _G) * _R, _R), :]
            hh = jnp.maximum(h * a + c, 0.0) * ns
            o_ref[...] = jnp.dot(hh, w_ref[...],
                                 preferred_element_type=jnp.float32)

    return pl.pallas_call(
        body,
        grid=(2 * _G,),
        in_specs=[
            pl.BlockSpec((NC, _R, D),
                         lambda i: (0, jnp.where(i < _G, i, _G - 1), 0)),
            pl.BlockSpec((NC, 2, _R), lambda i: (0, 0, i % _G)),
            pl.BlockSpec((1, D), lambda i: (0, 0)),
            pl.BlockSpec((1, D), lambda i: (0, 0)),
            pl.BlockSpec((1, D), lambda i: (0, 0)),
            pl.BlockSpec((D, D), lambda i: (0, 0)),
        ],
        out_specs=pl.BlockSpec((_R, D), lambda i: (i % _G, 0)),
        out_shape=jax.ShapeDtypeStruct((NP, D), jnp.float32),
        scratch_shapes=[pltpu.VMEM((NP, D), jnp.float32),
                        pltpu.VMEM((2, D), jnp.float32)],
    )(p, degp, b1, gamma, beta, w)


def _tc_final(q, degp, b2):
    def body(q_ref, dp_ref, b_ref, o_ref):
        nd = _nrm(dp_ref[0, 1] + dp_ref[1, 1]).reshape(_R, 1)
        o_ref[...] = (q_ref[0] + q_ref[1]) * nd + b_ref[...]

    return pl.pallas_call(
        body,
        grid=(_G,),
        in_specs=[pl.BlockSpec((NC, _R, D), lambda i: (0, i, 0)),
                  _DEG_SPEC,
                  pl.BlockSpec((1, D), lambda i: (0, 0))],
        out_specs=pl.BlockSpec((_R, D), lambda i: (i, 0)),
        out_shape=jax.ShapeDtypeStruct((N, D), jnp.float32),
    )(q, degp, b2)


def kernel(x, edge_index, W1, b1, gamma, beta, W2, b2):
    e6 = edge_index.reshape(2, NC, NS, NBLK, BCH, C)
    zeros = jnp.zeros((RPS, D), jnp.float32)
    zeros1 = jnp.zeros((RPS,), jnp.float32)
    x_p = jnp.concatenate(
        [x, jnp.zeros((NP - N, D), jnp.float32)], axis=0)

    u = _tc_mm(x_p, W1)
    degp = _sc_degrees(e6, zeros1)
    h1 = _tc_norm_scale(degp, u)

    p = _sc_edge_pass(h1, e6, zeros)
    h2 = _tc_layer2(p, degp, b1.reshape(1, D), gamma.reshape(1, D),
                    beta.reshape(1, D), W2)
    q = _sc_edge_pass(h2, e6, zeros)
    return _tc_final(q, degp, b2.reshape(1, D))
